# baseline (speedup 1.0000x reference)
import functools

import jax
import jax.numpy as jnp
import numpy as np
from jax import lax
from jax.experimental import pallas as pl
from jax.experimental.pallas import tpu as pltpu
from jax.experimental.pallas import tpu_sc as plsc

N_TGT_K = 10000
N_NBR_K = 100000
DEG_K = 32
E_K = N_TGT_K * DEG_K
D_K = 128

NW = 32
TPW = 320
CHUNK_T = 4
CHUNK_E = CHUNK_T * DEG_K
NCHUNK = TPW // CHUNK_T
PAD_T = NW * TPW
PAD_E = PAD_T * DEG_K

_mesh = plsc.VectorSubcoreMesh(core_axis_name="c", subcore_axis_name="s")


_NV = D_K // 16


@functools.partial(
    pl.kernel,
    out_type=jax.ShapeDtypeStruct((PAD_T, D_K), jnp.float32),
    mesh=_mesh,
    scratch_types=[
        pltpu.VMEM((NCHUNK, CHUNK_E), jnp.int32),
        pltpu.VMEM((CHUNK_E, D_K), jnp.float32),
        pltpu.VMEM((CHUNK_E, D_K), jnp.float32),
        pltpu.VMEM((CHUNK_E, D_K), jnp.float32),
        pltpu.VMEM((CHUNK_E, D_K), jnp.float32),
        pltpu.VMEM((TPW, D_K), jnp.float32),
        pltpu.SemaphoreType.DMA,
        pltpu.SemaphoreType.DMA,
        pltpu.SemaphoreType.DMA,
        pltpu.SemaphoreType.DMA,
    ],
)
def _sc_gather_sum(col_hbm, table_hbm, out_hbm, idx_v, rows0_v, rows1_v,
                   rows2_v, rows3_v, acc_v, sem0, sem1, sem2, sem3):
    wid = lax.axis_index("s") * 2 + lax.axis_index("c")
    pltpu.sync_copy(col_hbm.at[pl.ds(wid * NCHUNK, NCHUNK)], idx_v)

    def start(c, buf, sem):
        pltpu.async_copy(table_hbm.at[idx_v.at[c]], buf, sem)

    def wait(buf, sem):
        pltpu.make_async_copy(table_hbm.at[idx_v.at[0]], buf, sem).wait()

    def accum(buf, c):
        base = c * CHUNK_T
        for t in range(CHUNK_T):
            r0 = t * DEG_K
            init = tuple(buf[r0, pl.ds(d * 16, 16)] for d in range(_NV))

            def rbody(r, vs):
                return tuple(
                    vs[d] + buf[r0 + r, pl.ds(d * 16, 16)] for d in range(_NV)
                )

            vs = lax.fori_loop(1, DEG_K, rbody, init, unroll=4)
            for d in range(_NV):
                acc_v[base + t, pl.ds(d * 16, 16)] = vs[d]

    bufs = (rows0_v, rows1_v, rows2_v, rows3_v)
    sems = (sem0, sem1, sem2, sem3)
    NB = 4
    for k in range(NB):
        start(k, bufs[k], sems[k])

    def gbody(g, _):
        for k in range(NB):
            c = NB * g + k
            wait(bufs[k], sems[k])
            accum(bufs[k], c)

            @pl.when(g < NCHUNK // NB - 1)
            def _():
                start(c + NB, bufs[k], sems[k])

        return 0

    lax.fori_loop(0, NCHUNK // NB, gbody, 0)
    pltpu.sync_copy(acc_v, out_hbm.at[pl.ds(wid * TPW, TPW)])


def _mm_body(xt_ref, xs_ref, w1_ref, w2_ref, b_ref, o_ref):
    xs = xs_ref[...] * np.float32(1.0 / DEG_K)
    acc = jnp.dot(xt_ref[...], w1_ref[...], preferred_element_type=jnp.float32)
    acc = acc + jnp.dot(xs, w2_ref[...], preferred_element_type=jnp.float32)
    o_ref[...] = acc + b_ref[...]


_ROWS_BLK = 1000

_tc_linear = pl.pallas_call(
    _mm_body,
    grid=(N_TGT_K // _ROWS_BLK,),
    in_specs=[
        pl.BlockSpec((_ROWS_BLK, D_K), lambda i: (i, 0)),
        pl.BlockSpec((_ROWS_BLK, D_K), lambda i: (i, 0)),
        pl.BlockSpec((D_K, D_K), lambda i: (0, 0)),
        pl.BlockSpec((D_K, D_K), lambda i: (0, 0)),
        pl.BlockSpec((1, D_K), lambda i: (0, 0)),
    ],
    out_specs=pl.BlockSpec((_ROWS_BLK, D_K), lambda i: (i, 0)),
    out_shape=jax.ShapeDtypeStruct((N_TGT_K, D_K), jnp.float32),
)


def kernel(csr_row_ptr, csr_col_ind, sample_count, x_neighboor, x_target, W, b_lin, bias_param):
    col = csr_col_ind.astype(jnp.int32)
    col = jnp.concatenate([col, jnp.zeros((PAD_E - E_K,), jnp.int32)])
    col2d = col.reshape(NW * NCHUNK, CHUNK_E)
    xsum = _sc_gather_sum(col2d, x_neighboor)
    w1t = W[:, :D_K].T
    w2t = W[:, D_K:].T
    bvec = (b_lin + bias_param).reshape(1, D_K)
    return _tc_linear(x_target, xsum, w1t, w2t, bvec)
